# baseline (speedup 1.0000x reference)
import jax
import jax.numpy as jnp
from jax.experimental import pallas as pl
from jax.experimental.pallas import tpu as pltpu

T = 128
N = 512
EMB = 16
PE_DIM = 128
OUT_DIM = PE_DIM + EMB
TC_CHUNK = 16


def _side_info_block(w_ref, out_ref):
    tci = pl.program_id(1)
    t0 = (tci * TC_CHUNK).astype(jnp.float32)
    t = t0 + jax.lax.broadcasted_iota(
        jnp.int32, (TC_CHUNK, PE_DIM), 0
    ).astype(jnp.float32)
    c = jax.lax.broadcasted_iota(jnp.int32, (TC_CHUNK, PE_DIM), 1)
    pair = (c // 2).astype(jnp.float32)
    div = jnp.exp(pair * (-2.0 * jnp.log(10000.0) / PE_DIM))
    ang = t * div
    pe = jnp.where(c % 2 == 0, jnp.sin(ang), jnp.cos(ang))
    out_ref[0, :, :, 0:PE_DIM] = jnp.broadcast_to(
        pe[:, None, :], (TC_CHUNK, N, PE_DIM)
    )
    out_ref[0, :, :, PE_DIM:OUT_DIM] = jnp.broadcast_to(
        w_ref[...][None, :, :], (TC_CHUNK, N, EMB)
    )


def kernel(u, W):
    batch = u.shape[0]
    grid = (batch, T // TC_CHUNK)
    return pl.pallas_call(
        _side_info_block,
        grid=grid,
        in_specs=[pl.BlockSpec((N, EMB), lambda b, tc: (0, 0))],
        out_specs=pl.BlockSpec(
            (1, TC_CHUNK, N, OUT_DIM), lambda b, tc: (b, tc, 0, 0)
        ),
        out_shape=jax.ShapeDtypeStruct((batch, T, N, OUT_DIM), jnp.float32),
        compiler_params=pltpu.CompilerParams(
            dimension_semantics=("parallel", "parallel"),
        ),
    )(W)


# speedup vs baseline: 6.0138x; 6.0138x over previous
import jax
import jax.numpy as jnp
from jax.experimental import pallas as pl
from jax.experimental.pallas import tpu as pltpu

T = 128
N = 512
EMB = 16
PE_DIM = 128
OUT_DIM = PE_DIM + EMB
TC_CHUNK = 16


def _side_info_block(wt_ref, out_ref):
    tci = pl.program_id(1)
    t0 = (tci * TC_CHUNK).astype(jnp.float32)
    c = jax.lax.broadcasted_iota(jnp.int32, (PE_DIM, TC_CHUNK), 0)
    t = t0 + jax.lax.broadcasted_iota(
        jnp.int32, (PE_DIM, TC_CHUNK), 1
    ).astype(jnp.float32)
    pair = (c // 2).astype(jnp.float32)
    div = jnp.exp(pair * (-2.0 * jnp.log(10000.0) / PE_DIM))
    ang = t * div
    val = jnp.where(c % 2 == 0, jnp.sin(ang), jnp.cos(ang))
    wt = wt_ref[...]
    for ti in range(TC_CHUNK):
        out_ref[0, ti, 0:PE_DIM, :] = jnp.broadcast_to(
            val[:, ti : ti + 1], (PE_DIM, N)
        )
        out_ref[0, ti, PE_DIM:OUT_DIM, :] = wt


def kernel(u, W):
    batch = u.shape[0]
    grid = (batch, T // TC_CHUNK)
    out = pl.pallas_call(
        _side_info_block,
        grid=grid,
        in_specs=[pl.BlockSpec((EMB, N), lambda b, tc: (0, 0))],
        out_specs=pl.BlockSpec(
            (1, TC_CHUNK, OUT_DIM, N), lambda b, tc: (b, tc, 0, 0)
        ),
        out_shape=jax.ShapeDtypeStruct((batch, T, OUT_DIM, N), jnp.float32),
        compiler_params=pltpu.CompilerParams(
            dimension_semantics=("parallel", "parallel"),
        ),
    )(W.T)
    return jnp.transpose(out, (0, 1, 3, 2))
